# baseline (speedup 1.0000x reference)
import functools

import jax
import jax.numpy as jnp
from jax.experimental import pallas as pl
from jax.experimental.pallas import tpu as pltpu

F32 = jnp.float32
DEPTH = 4
NEXP = 8
TOPK = 2
DM = 768
NH = 12
PS = 16
IMG = 224
NC = 1000
MOE_H = 576
MLP_H = 3072
NPAT = 196
BN = 16
NTOK = 197
T = BN * NTOK
DH = DM // NH


def _ln2d(x, g, b):
    m = jnp.mean(x, axis=-1, keepdims=True)
    v = jnp.mean((x - m) ** 2, axis=-1, keepdims=True)
    return (x - m) / jnp.sqrt(v + 1e-5) * g + b


def _dot(a, b):
    return jax.lax.dot_general(a, b, (((1,), (0,)), ((), ())),
                               preferred_element_type=F32,
                               precision=jax.lax.Precision.HIGHEST)


def _gelu(x):
    return 0.5 * x * (1.0 + jax.lax.erf(x * (2.0 ** -0.5)))


def _patch_body(xp_ref, w_ref, b_ref, pos_ref, o_ref):
    x = xp_ref[0]
    o_ref[0] = _dot(x, w_ref[...]) + b_ref[...] + pos_ref[0]


def _patch_embed(xp, w, bias, pos):
    return pl.pallas_call(
        _patch_body,
        grid=(BN,),
        in_specs=[
            pl.BlockSpec((1, NPAT, 768), lambda i: (i, 0, 0)),
            pl.BlockSpec((768, DM), lambda i: (0, 0)),
            pl.BlockSpec((1, DM), lambda i: (0, 0)),
            pl.BlockSpec((1, NPAT, DM), lambda i: (0, 0, 0)),
        ],
        out_specs=pl.BlockSpec((1, NPAT, DM), lambda i: (i, 0, 0)),
        out_shape=jax.ShapeDtypeStruct((BN, NPAT, DM), F32),
    )(xp, w, bias, pos)


def _attn_body(x_ref, g_ref, b_ref, qw_ref, qb_ref, pw_ref, pb_ref, o_ref):
    x = x_ref[0]
    xn = _ln2d(x, g_ref[...], b_ref[...])
    qkv = _dot(xn, qw_ref[...]) + qb_ref[...]
    outs = []
    scale = DH ** -0.5
    for h in range(NH):
        q = qkv[:, h * DH:(h + 1) * DH]
        k = qkv[:, DM + h * DH: DM + (h + 1) * DH]
        v = qkv[:, 2 * DM + h * DH: 2 * DM + (h + 1) * DH]
        s = jax.lax.dot_general(q, k, (((1,), (1,)), ((), ())),
                                preferred_element_type=F32,
                                precision=jax.lax.Precision.HIGHEST) * scale
        p = jax.nn.softmax(s, axis=-1)
        outs.append(_dot(p, v))
    att = jnp.concatenate(outs, axis=1)
    o_ref[0] = x + _dot(att, pw_ref[...]) + pb_ref[...]


def _attn_block(x, g, b, qw, qb, pw, pb):
    return pl.pallas_call(
        _attn_body,
        grid=(BN,),
        in_specs=[
            pl.BlockSpec((1, NTOK, DM), lambda i: (i, 0, 0)),
            pl.BlockSpec((1, DM), lambda i: (0, 0)),
            pl.BlockSpec((1, DM), lambda i: (0, 0)),
            pl.BlockSpec((DM, 3 * DM), lambda i: (0, 0)),
            pl.BlockSpec((1, 3 * DM), lambda i: (0, 0)),
            pl.BlockSpec((DM, DM), lambda i: (0, 0)),
            pl.BlockSpec((1, DM), lambda i: (0, 0)),
        ],
        out_specs=pl.BlockSpec((1, NTOK, DM), lambda i: (i, 0, 0)),
        out_shape=jax.ShapeDtypeStruct((BN, NTOK, DM), F32),
    )(x, g, b, qw, qb, pw, pb)


def _mlp_body(x_ref, g_ref, b_ref, w1_ref, b1_ref, w2_ref, b2_ref, o_ref):
    x = x_ref[...]
    xn = _ln2d(x, g_ref[...], b_ref[...])
    h = _gelu(_dot(xn, w1_ref[...]) + b1_ref[...])
    o_ref[...] = x + _dot(h, w2_ref[...]) + b2_ref[...]


def _mlp_block(x2d, g, b, w1, b1, w2, b2):
    TB = 394 * 2
    TB = 512
    nb = pl.cdiv(T, TB)
    return pl.pallas_call(
        _mlp_body,
        grid=(nb,),
        in_specs=[
            pl.BlockSpec((TB, DM), lambda i: (i, 0)),
            pl.BlockSpec((1, DM), lambda i: (0, 0)),
            pl.BlockSpec((1, DM), lambda i: (0, 0)),
            pl.BlockSpec((DM, MLP_H), lambda i: (0, 0)),
            pl.BlockSpec((1, MLP_H), lambda i: (0, 0)),
            pl.BlockSpec((MLP_H, DM), lambda i: (0, 0)),
            pl.BlockSpec((1, DM), lambda i: (0, 0)),
        ],
        out_specs=pl.BlockSpec((TB, DM), lambda i: (i, 0)),
        out_shape=jax.ShapeDtypeStruct((T, DM), F32),
    )(x2d, g, b, w1, b1, w2, b2)


RTB = 512


def _router_body(x_ref, g_ref, b_ref, rw_ref, xf_ref, gates_ref, aux_ref,
                 psum, csum):
    i = pl.program_id(0)
    nb = pl.num_programs(0)
    x = x_ref[...]
    xn = _ln2d(x, g_ref[...], b_ref[...])
    xf_ref[...] = xn
    logits = _dot(xn, rw_ref[...])
    probs = jax.nn.softmax(logits, axis=-1)
    rows = jax.lax.broadcasted_iota(jnp.int32, (RTB, 1), 0) + i * RTB
    valid = rows < T
    lane = jax.lax.broadcasted_iota(jnp.int32, (RTB, NEXP), 1)
    m0 = jnp.max(probs, axis=-1, keepdims=True)
    i0 = jnp.min(jnp.where(probs == m0, lane, NEXP), axis=-1, keepdims=True)
    pm = jnp.where(lane == i0, -jnp.inf, probs)
    m1 = jnp.max(pm, axis=-1, keepdims=True)
    i1 = jnp.min(jnp.where(pm == m1, lane, NEXP), axis=-1, keepdims=True)
    ssum = m0 + m1
    sel0 = lane == i0
    sel1 = lane == i1
    gates = jnp.where(sel0, m0 / ssum, 0.0) + jnp.where(sel1, m1 / ssum, 0.0)
    gates_ref[...] = gates
    pblk = jnp.sum(jnp.where(valid, probs, 0.0), axis=0, keepdims=True)
    cblk = jnp.sum(jnp.where(valid & (sel0 | sel1), 1.0, 0.0), axis=0,
                   keepdims=True)

    @pl.when(i == 0)
    def _():
        psum[...] = jnp.zeros_like(psum)
        csum[...] = jnp.zeros_like(csum)

    psum[...] += pblk
    csum[...] += cblk

    @pl.when(i == nb - 1)
    def _():
        fe = csum[...] / (T * TOPK)
        pe = psum[...] / T
        aux_ref[...] = NEXP * jnp.sum(fe * pe, axis=-1, keepdims=True)


def _router(x2d, g, b, rw):
    nb = pl.cdiv(T, RTB)
    return pl.pallas_call(
        _router_body,
        grid=(nb,),
        in_specs=[
            pl.BlockSpec((RTB, DM), lambda i: (i, 0)),
            pl.BlockSpec((1, DM), lambda i: (0, 0)),
            pl.BlockSpec((1, DM), lambda i: (0, 0)),
            pl.BlockSpec((DM, NEXP), lambda i: (0, 0)),
        ],
        out_specs=[
            pl.BlockSpec((RTB, DM), lambda i: (i, 0)),
            pl.BlockSpec((RTB, NEXP), lambda i: (i, 0)),
            pl.BlockSpec((1, 1), lambda i: (0, 0)),
        ],
        out_shape=[
            jax.ShapeDtypeStruct((T, DM), F32),
            jax.ShapeDtypeStruct((T, NEXP), F32),
            jax.ShapeDtypeStruct((1, 1), F32),
        ],
        scratch_shapes=[pltpu.VMEM((1, NEXP), F32), pltpu.VMEM((1, NEXP), F32)],
    )(x2d, g, b, rw)


ETB = 800


def _moe_dense_body(xf_ref, gates_ref, x_ref, w1_ref, b1_ref, w2_ref, b2_ref,
                    o_ref):
    e = pl.program_id(1)

    @pl.when(e == 0)
    def _():
        o_ref[...] = x_ref[...]

    xf = xf_ref[...]
    h = _gelu(_dot(xf, w1_ref[0]) + b1_ref[0])
    eo = _dot(h, w2_ref[0]) + b2_ref[0]
    lane = jax.lax.broadcasted_iota(jnp.int32, (ETB, NEXP), 1)
    gcol = jnp.sum(jnp.where(lane == e, gates_ref[...], 0.0), axis=-1,
                   keepdims=True)
    o_ref[...] += gcol * eo


def _moe_dense(xf, gates, x2d, w1, b1, w2, b2):
    nb = pl.cdiv(T, ETB)
    return pl.pallas_call(
        _moe_dense_body,
        grid=(nb, NEXP),
        in_specs=[
            pl.BlockSpec((ETB, DM), lambda t, e: (t, 0)),
            pl.BlockSpec((ETB, NEXP), lambda t, e: (t, 0)),
            pl.BlockSpec((ETB, DM), lambda t, e: (t, 0)),
            pl.BlockSpec((1, DM, MOE_H), lambda t, e: (e, 0, 0)),
            pl.BlockSpec((1, 1, MOE_H), lambda t, e: (e, 0, 0)),
            pl.BlockSpec((1, MOE_H, DM), lambda t, e: (e, 0, 0)),
            pl.BlockSpec((1, 1, DM), lambda t, e: (e, 0, 0)),
        ],
        out_specs=pl.BlockSpec((ETB, DM), lambda t, e: (t, 0)),
        out_shape=jax.ShapeDtypeStruct((T, DM), F32),
    )(xf, gates, x2d, w1, b1, w2, b2)


def _head_body(x_ref, g_ref, b_ref, w_ref, bias_ref, o_ref):
    x = x_ref[...]
    feat = jnp.mean(x[:, 1:, :], axis=1)
    feat = _ln2d(feat, g_ref[...], b_ref[...])
    o_ref[...] = _dot(feat, w_ref[...]) + bias_ref[...]


def _head(x3d, g, b, w, bias):
    return pl.pallas_call(
        _head_body,
        grid=(1,),
        in_specs=[
            pl.BlockSpec((BN, NTOK, DM), lambda i: (0, 0, 0)),
            pl.BlockSpec((1, DM), lambda i: (0, 0)),
            pl.BlockSpec((1, DM), lambda i: (0, 0)),
            pl.BlockSpec((DM, NC), lambda i: (0, 0)),
            pl.BlockSpec((1, NC), lambda i: (0, 0)),
        ],
        out_specs=pl.BlockSpec((BN, NC), lambda i: (0, 0)),
        out_shape=jax.ShapeDtypeStruct((BN, NC), F32),
    )(x3d, g, b, w, bias)


def kernel(imgs, conv_w, conv_b, cls_token, pos_embed, b0_ln1_g, b0_ln1_b,
           b0_qkv_w, b0_qkv_b, b0_proj_w, b0_proj_b, b0_ln2_g, b0_ln2_b,
           b0_fc1_w, b0_fc1_b, b0_fc2_w, b0_fc2_b, m_ln1_g, m_ln1_b, m_qkv_w,
           m_qkv_b, m_proj_w, m_proj_b, m_ln2_g, m_ln2_b, m_router_w, m_e_w1,
           m_e_b1, m_e_w2, m_e_b2, fc_norm_g, fc_norm_b, head_w, head_b):
    r2 = lambda a: a.reshape(1, -1)
    xp = imgs.reshape(BN, 3, IMG // PS, PS, IMG // PS, PS)
    xp = xp.transpose(0, 2, 4, 1, 3, 5).reshape(BN, NPAT, 3 * PS * PS)
    w = conv_w.reshape(DM, -1).T
    x = _patch_embed(xp, w, r2(conv_b), pos_embed[:, 1:, :])
    cls = (cls_token + pos_embed[:, :1, :]).astype(F32)
    x = jnp.concatenate([jnp.broadcast_to(cls, (BN, 1, DM)), x], axis=1)

    x = _attn_block(x, r2(b0_ln1_g), r2(b0_ln1_b), b0_qkv_w, r2(b0_qkv_b),
                    b0_proj_w, r2(b0_proj_b))
    x2d = x.reshape(T, DM)
    x2d = _mlp_block(x2d, r2(b0_ln2_g), r2(b0_ln2_b), b0_fc1_w, r2(b0_fc1_b),
                     b0_fc2_w, r2(b0_fc2_b))
    x = x2d.reshape(BN, NTOK, DM)

    auxs = []
    for i in range(DEPTH - 1):
        x = _attn_block(x, r2(m_ln1_g[i]), r2(m_ln1_b[i]), m_qkv_w[i],
                        r2(m_qkv_b[i]), m_proj_w[i], r2(m_proj_b[i]))
        x2d = x.reshape(T, DM)
        xf, gates, aux = _router(x2d, r2(m_ln2_g[i]), r2(m_ln2_b[i]),
                                 m_router_w[i])
        x2d = _moe_dense(xf, gates, x2d, m_e_w1[i],
                         m_e_b1[i].reshape(NEXP, 1, MOE_H), m_e_w2[i],
                         m_e_b2[i].reshape(NEXP, 1, DM))
        x = x2d.reshape(BN, NTOK, DM)
        auxs.append(aux[0, 0])

    logits = _head(x, r2(fc_norm_g), r2(fc_norm_b), head_w, r2(head_b))
    return logits, jnp.stack(auxs)
